# baseline (speedup 1.0000x reference)
import functools

import jax
import jax.numpy as jnp
from jax import lax
from jax.experimental import pallas as pl
from jax.experimental.pallas import tpu as pltpu
from jax.experimental.pallas import tpu_sc as plsc

BT = 32
CH = 512


def _router_body(x_ref, gw_ref, p_ref, be_ref, oh_ref, cs_ref):
    T = x_ref.shape[0]
    E = gw_ref.shape[0]
    NBLK = be_ref.shape[0]
    logits = lax.dot_general(
        x_ref[...], gw_ref[...], (((1,), (1,)), ((), ())),
        preferred_element_type=jnp.float32)
    m = jnp.max(logits, axis=1, keepdims=True)
    col = lax.broadcasted_iota(jnp.int32, logits.shape, 1)
    assign = jnp.min(jnp.where(logits >= m, col, E), axis=1, keepdims=True)
    oh_ref[...] = (col == assign).astype(jnp.float32)

    r_i = lax.broadcasted_iota(jnp.int32, (CH, CH), 0)
    c_i = lax.broadcasted_iota(jnp.int32, (CH, CH), 1)
    tril = (c_i < r_i).astype(jnp.float32)

    def chunk(i, tot):
        oh = oh_ref[pl.ds(i * CH, CH), :]
        cs_ref[pl.ds(i * CH, CH), :] = lax.dot_general(
            tril, oh, (((1,), (0,)), ((), ())),
            preferred_element_type=jnp.float32) + tot
        return tot + jnp.sum(oh, axis=0, keepdims=True)

    counts = lax.fori_loop(0, T // CH, chunk, jnp.zeros((1, E), jnp.float32))

    blocks = jnp.floor((counts + (BT - 1)) * (1.0 / BT))
    e_r = lax.broadcasted_iota(jnp.int32, (E, E), 0)
    e_c = lax.broadcasted_iota(jnp.int32, (E, E), 1)
    incl = (e_r <= e_c).astype(jnp.float32)
    cumb = lax.dot_general(blocks, incl, (((1,), (0,)), ((), ())),
                           preferred_element_type=jnp.float32)
    pad_start = (cumb - blocks) * BT

    oh = oh_ref[...]
    rank = jnp.sum(cs_ref[...] * oh, axis=1, keepdims=True)
    p_ref[...] = (rank + jnp.sum(oh * pad_start, axis=1,
                                 keepdims=True)).astype(jnp.int32)

    bb = lax.broadcasted_iota(jnp.int32, (NBLK, E), 0).astype(jnp.float32)
    be = jnp.sum((cumb <= bb).astype(jnp.int32), axis=1, keepdims=True)
    be_ref[...] = jnp.minimum(be, E - 1)


def _mlp_body(be_ref, xs_ref, g1_ref, g2_ref, u1_ref, u2_ref, d1_ref, d2_ref,
              ys_ref):
    x = xs_ref[...]

    def half(g_ref, u_ref):
        g = lax.dot_general(x, g_ref[0], (((1,), (1,)), ((), ())),
                            preferred_element_type=jnp.float32)
        u = lax.dot_general(x, u_ref[0], (((1,), (1,)), ((), ())),
                            preferred_element_type=jnp.float32)
        return (g * jax.nn.sigmoid(g)) * u

    h = jnp.concatenate([half(g1_ref, u1_ref), half(g2_ref, u2_ref)], axis=1)
    H2 = d1_ref.shape[1]
    ys_ref[:, :H2] = lax.dot_general(h, d1_ref[0], (((1,), (1,)), ((), ())),
                                     preferred_element_type=jnp.float32)
    ys_ref[:, H2:] = lax.dot_general(h, d2_ref[0], (((1,), (1,)), ((), ())),
                                     preferred_element_type=jnp.float32)


def kernel(hidden_states, gate_w, gate_proj, up_proj, down_proj):
    Bt, St, H = hidden_states.shape
    E, I, _ = gate_proj.shape
    T = Bt * St
    NBLK = T // BT + E
    NPAD = NBLK * BT

    info = plsc.get_sparse_core_info()
    NC = info.num_cores
    NW = NC * info.num_subcores
    RPW = T // NW

    x2d = hidden_states.reshape(T, H)

    p_out, be_out = pl.pallas_call(
        _router_body,
        out_shape=[jax.ShapeDtypeStruct((T, 1), jnp.int32),
                   jax.ShapeDtypeStruct((NBLK, 1), jnp.int32)],
        scratch_shapes=[pltpu.VMEM((T, E), jnp.float32),
                        pltpu.VMEM((T, E), jnp.float32)],
    )(x2d, gate_w)
    block_expert = be_out[:, 0]
    p2 = p_out.reshape(NW, RPW)

    mesh = plsc.VectorSubcoreMesh(core_axis_name="c", subcore_axis_name="s")

    @functools.partial(
        pl.kernel, mesh=mesh,
        out_type=jax.ShapeDtypeStruct((NPAD, H), jnp.float32),
        scratch_types=[pltpu.VMEM((RPW,), jnp.int32),
                       pltpu.VMEM((RPW, H), jnp.float32),
                       pltpu.SemaphoreType.DMA])
    def _dispatch(x_hbm, idx_hbm, xs_hbm, idx_v, rows_v, sem):
        wid = lax.axis_index("s") * NC + lax.axis_index("c")
        pltpu.sync_copy(idx_hbm.at[wid], idx_v)
        pltpu.sync_copy(x_hbm.at[pl.ds(wid * RPW, RPW)], rows_v)
        pltpu.async_copy(rows_v, xs_hbm.at[idx_v], sem).wait()

    xs = _dispatch(x2d, p2)

    I2 = I // 2
    grid_spec = pltpu.PrefetchScalarGridSpec(
        num_scalar_prefetch=1,
        grid=(NBLK,),
        in_specs=[
            pl.BlockSpec((BT, H), lambda b, be: (b, 0)),
            pl.BlockSpec((1, I2, H), lambda b, be: (be[b], 0, 0)),
            pl.BlockSpec((1, I2, H), lambda b, be: (be[b], 1, 0)),
            pl.BlockSpec((1, I2, H), lambda b, be: (be[b], 0, 0)),
            pl.BlockSpec((1, I2, H), lambda b, be: (be[b], 1, 0)),
            pl.BlockSpec((1, H // 2, I), lambda b, be: (be[b], 0, 0)),
            pl.BlockSpec((1, H // 2, I), lambda b, be: (be[b], 1, 0)),
        ],
        out_specs=pl.BlockSpec((BT, H), lambda b, be: (b, 0)),
    )
    ys = pl.pallas_call(
        _mlp_body,
        grid_spec=grid_spec,
        out_shape=jax.ShapeDtypeStruct((NPAD, H), jnp.float32),
    )(block_expert, xs, gate_proj, gate_proj, up_proj, up_proj,
      down_proj, down_proj)

    @functools.partial(
        pl.kernel, mesh=mesh,
        out_type=jax.ShapeDtypeStruct((T, H), jnp.float32),
        scratch_types=[pltpu.VMEM((RPW,), jnp.int32),
                       pltpu.VMEM((RPW, H), jnp.float32),
                       pltpu.SemaphoreType.DMA])
    def _combine(ys_hbm, idx_hbm, out_hbm, idx_v, rows_v, sem):
        wid = lax.axis_index("s") * NC + lax.axis_index("c")
        pltpu.sync_copy(idx_hbm.at[wid], idx_v)
        pltpu.async_copy(ys_hbm.at[idx_v], rows_v, sem).wait()
        pltpu.sync_copy(rows_v, out_hbm.at[pl.ds(wid * RPW, RPW)])

    out = _combine(ys, p2)
    return out.reshape(Bt, St, H)


# speedup vs baseline: 1.2472x; 1.2472x over previous
import functools

import jax
import jax.numpy as jnp
from jax import lax
from jax.experimental import pallas as pl
from jax.experimental.pallas import tpu as pltpu
from jax.experimental.pallas import tpu_sc as plsc

BT = 128
CH = 512


def _router_body(x_ref, gw_ref, p_ref, be_ref, oh_ref, cs_ref):
    T = x_ref.shape[0]
    E = gw_ref.shape[0]
    NBLK = be_ref.shape[0]
    logits = lax.dot_general(
        x_ref[...], gw_ref[...], (((1,), (1,)), ((), ())),
        preferred_element_type=jnp.float32)
    m = jnp.max(logits, axis=1, keepdims=True)
    col = lax.broadcasted_iota(jnp.int32, logits.shape, 1)
    assign = jnp.min(jnp.where(logits >= m, col, E), axis=1, keepdims=True)
    oh_ref[...] = (col == assign).astype(jnp.float32)

    r_i = lax.broadcasted_iota(jnp.int32, (CH, CH), 0)
    c_i = lax.broadcasted_iota(jnp.int32, (CH, CH), 1)
    tril = (c_i < r_i).astype(jnp.float32)

    def chunk(i, tot):
        oh = oh_ref[pl.ds(i * CH, CH), :]
        cs_ref[pl.ds(i * CH, CH), :] = lax.dot_general(
            tril, oh, (((1,), (0,)), ((), ())),
            preferred_element_type=jnp.float32) + tot
        return tot + jnp.sum(oh, axis=0, keepdims=True)

    counts = lax.fori_loop(0, T // CH, chunk, jnp.zeros((1, E), jnp.float32))

    blocks = jnp.floor((counts + (BT - 1)) * (1.0 / BT))
    e_r = lax.broadcasted_iota(jnp.int32, (E, E), 0)
    e_c = lax.broadcasted_iota(jnp.int32, (E, E), 1)
    incl = (e_r <= e_c).astype(jnp.float32)
    cumb = lax.dot_general(blocks, incl, (((1,), (0,)), ((), ())),
                           preferred_element_type=jnp.float32)
    pad_start = (cumb - blocks) * BT

    oh = oh_ref[...]
    rank = jnp.sum(cs_ref[...] * oh, axis=1, keepdims=True)
    p_ref[...] = (rank + jnp.sum(oh * pad_start, axis=1,
                                 keepdims=True)).astype(jnp.int32)

    bb = lax.broadcasted_iota(jnp.int32, (NBLK, E), 0).astype(jnp.float32)
    be = jnp.sum((cumb <= bb).astype(jnp.int32), axis=1, keepdims=True)
    be_ref[...] = jnp.minimum(be, E - 1)


def _mlp_body(be_ref, xs_ref, g1_ref, g2_ref, u1_ref, u2_ref, d1_ref, d2_ref,
              ys_ref):
    x = xs_ref[...]

    def half(g_ref, u_ref):
        g = lax.dot_general(x, g_ref[0], (((1,), (1,)), ((), ())),
                            preferred_element_type=jnp.float32)
        u = lax.dot_general(x, u_ref[0], (((1,), (1,)), ((), ())),
                            preferred_element_type=jnp.float32)
        return (g * jax.nn.sigmoid(g)) * u

    h = jnp.concatenate([half(g1_ref, u1_ref), half(g2_ref, u2_ref)], axis=1)
    H2 = d1_ref.shape[1]
    ys_ref[:, :H2] = lax.dot_general(h, d1_ref[0], (((1,), (1,)), ((), ())),
                                     preferred_element_type=jnp.float32)
    ys_ref[:, H2:] = lax.dot_general(h, d2_ref[0], (((1,), (1,)), ((), ())),
                                     preferred_element_type=jnp.float32)


def kernel(hidden_states, gate_w, gate_proj, up_proj, down_proj):
    Bt, St, H = hidden_states.shape
    E, I, _ = gate_proj.shape
    T = Bt * St
    NBLK = T // BT + E
    NPAD = NBLK * BT

    info = plsc.get_sparse_core_info()
    NC = info.num_cores
    NW = NC * info.num_subcores
    RPW = T // NW

    x2d = hidden_states.reshape(T, H)

    p_out, be_out = pl.pallas_call(
        _router_body,
        out_shape=[jax.ShapeDtypeStruct((T, 1), jnp.int32),
                   jax.ShapeDtypeStruct((NBLK, 1), jnp.int32)],
        scratch_shapes=[pltpu.VMEM((T, E), jnp.float32),
                        pltpu.VMEM((T, E), jnp.float32)],
    )(x2d, gate_w)
    block_expert = be_out[:, 0]
    p2 = p_out.reshape(NW, RPW)

    mesh = plsc.VectorSubcoreMesh(core_axis_name="c", subcore_axis_name="s")

    @functools.partial(
        pl.kernel, mesh=mesh,
        out_type=jax.ShapeDtypeStruct((NPAD, H), jnp.float32),
        scratch_types=[pltpu.VMEM((RPW,), jnp.int32),
                       pltpu.VMEM((RPW, H), jnp.float32),
                       pltpu.SemaphoreType.DMA])
    def _dispatch(x_hbm, idx_hbm, xs_hbm, idx_v, rows_v, sem):
        wid = lax.axis_index("s") * NC + lax.axis_index("c")
        pltpu.sync_copy(idx_hbm.at[wid], idx_v)
        pltpu.sync_copy(x_hbm.at[pl.ds(wid * RPW, RPW)], rows_v)
        pltpu.async_copy(rows_v, xs_hbm.at[idx_v], sem).wait()

    xs = _dispatch(x2d, p2)

    I2 = I // 2
    grid_spec = pltpu.PrefetchScalarGridSpec(
        num_scalar_prefetch=1,
        grid=(NBLK,),
        in_specs=[
            pl.BlockSpec((BT, H), lambda b, be: (b, 0)),
            pl.BlockSpec((1, I2, H), lambda b, be: (be[b], 0, 0)),
            pl.BlockSpec((1, I2, H), lambda b, be: (be[b], 1, 0)),
            pl.BlockSpec((1, I2, H), lambda b, be: (be[b], 0, 0)),
            pl.BlockSpec((1, I2, H), lambda b, be: (be[b], 1, 0)),
            pl.BlockSpec((1, H // 2, I), lambda b, be: (be[b], 0, 0)),
            pl.BlockSpec((1, H // 2, I), lambda b, be: (be[b], 1, 0)),
        ],
        out_specs=pl.BlockSpec((BT, H), lambda b, be: (b, 0)),
    )
    ys = pl.pallas_call(
        _mlp_body,
        grid_spec=grid_spec,
        out_shape=jax.ShapeDtypeStruct((NPAD, H), jnp.float32),
    )(block_expert, xs, gate_proj, gate_proj, up_proj, up_proj,
      down_proj, down_proj)

    @functools.partial(
        pl.kernel, mesh=mesh,
        out_type=jax.ShapeDtypeStruct((T, H), jnp.float32),
        scratch_types=[pltpu.VMEM((RPW,), jnp.int32),
                       pltpu.VMEM((RPW, H), jnp.float32),
                       pltpu.SemaphoreType.DMA])
    def _combine(ys_hbm, idx_hbm, out_hbm, idx_v, rows_v, sem):
        wid = lax.axis_index("s") * NC + lax.axis_index("c")
        pltpu.sync_copy(idx_hbm.at[wid], idx_v)
        pltpu.async_copy(ys_hbm.at[idx_v], rows_v, sem).wait()
        pltpu.sync_copy(rows_v, out_hbm.at[pl.ds(wid * RPW, RPW)])

    out = _combine(ys, p2)
    return out.reshape(Bt, St, H)
